# baseline (speedup 1.0000x reference)
import jax
import jax.numpy as jnp
from jax.experimental import pallas as pl

VOXEL = 0.005


def _mlp_body(f_ref, w1_ref, b1_ref, w2_ref, b2_ref, w3_ref, b3_ref, o_ref):
    h = jnp.maximum(f_ref[...] @ w1_ref[...] + b1_ref[...][None, :], 0.0)
    h = jnp.maximum(h @ w2_ref[...] + b2_ref[...][None, :], 0.0)
    o_ref[...] = h @ w3_ref[...] + b3_ref[...][None, :]


def _sample3d(vol, grid, dims):
    C = vol.shape[-1]
    D, H, W = dims[0], dims[1], dims[2]
    x = (grid[:, 0] + 1.0) * W * 0.5 - 0.5
    y = (grid[:, 1] + 1.0) * H * 0.5 - 0.5
    z = (grid[:, 2] + 1.0) * D * 0.5 - 0.5
    x0 = jnp.floor(x); y0 = jnp.floor(y); z0 = jnp.floor(z)
    wx = x - x0; wy = y - y0; wz = z - z0
    out = jnp.zeros((grid.shape[0], C), vol.dtype)
    for dz in (0, 1):
        for dy in (0, 1):
            for dx in (0, 1):
                zi = jnp.clip(z0 + dz, 0, D - 1).astype(jnp.int32)
                yi = jnp.clip(y0 + dy, 0, H - 1).astype(jnp.int32)
                xi = jnp.clip(x0 + dx, 0, W - 1).astype(jnp.int32)
                w = ((wz if dz else 1 - wz) * (wy if dy else 1 - wy) * (wx if dx else 1 - wx))[:, None]
                out = out + w * vol[zi, yi, xi]
    return out


def kernel(features, cnl_verts, canonical_pts, W1, b1, W2, b2, W3, b3, K1, K2, K3, K4):
    min_xyz = jnp.min(cnl_verts, axis=1) - 0.05
    max_xyz = jnp.max(cnl_verts, axis=1) + 0.05
    min_dhw = min_xyz[:, jnp.array([2, 1, 0])]
    max_dhw = max_xyz[:, jnp.array([2, 1, 0])]
    dhw = cnl_verts[..., jnp.array([2, 1, 0])]
    coord = jnp.round((dhw - min_dhw[:, None]) / VOXEL).astype(jnp.int32)
    out_sh = jnp.ceil((max_dhw - min_dhw) / VOXEL).astype(jnp.int32)
    out_sh = (out_sh | 31) + 1
    out_sh = jnp.max(out_sh, axis=0)
    PAD = 96

    x = pl.pallas_call(
        _mlp_body,
        out_shape=jax.ShapeDtypeStruct((features.shape[1], 32), jnp.float32),
    )(features[0], W1, b1, W2, b2, W3, b3)

    c = coord.reshape(-1, 3)
    vol = jnp.zeros((PAD, PAD, PAD, 32), jnp.float32).at[c[:, 0], c[:, 1], c[:, 2]].add(x)
    q = canonical_pts[..., jnp.array([2, 1, 0])] - min_dhw[:, None]
    q = q / VOXEL
    q = q / jnp.asarray(out_sh, jnp.float32) * 2.0 - 1.0
    grid = q[..., jnp.array([2, 1, 0])][0]
    v = vol[None]
    dims = out_sh
    feats = []
    for K in (K1, K2, K3, K4):
        v = jax.lax.conv_general_dilated(v, K, (2, 2, 2), 'SAME', dimension_numbers=('NDHWC', 'DHWIO', 'NDHWC'))
        v = jax.nn.relu(v)
        dims = dims // 2
        feats.append(_sample3d(v[0], grid, dims))
    return jnp.concatenate(feats, axis=-1)[None]


# speedup vs baseline: 1.0154x; 1.0154x over previous
import functools

import jax
import jax.numpy as jnp
from jax import lax
from jax.experimental import pallas as pl
from jax.experimental.pallas import tpu as pltpu
from jax.experimental.pallas import tpu_sc as plsc

VOXEL = 0.005
NC, NS = 2, 16
NTILES = NC * NS
NV = 6890
VT = 512
NG = 14
VP = NG * VT
NCONTRIB = 8 * VP
ROWS_PER_SUB = NCONTRIB // NS
L1DEP = 50
L1HW = 52
PLANE = L1HW * L1HW
CHUNK = 208
SHARD_PLANES = 16
SHARD_ROWS = SHARD_PLANES * PLANE
DUMMY = SHARD_ROWS
SHARD_ALL = SHARD_ROWS + 8


def _prep_body(f_ref, c_ref, w1_ref, b1_ref, w2_ref, b2_ref, w3_ref, b3_ref,
               k1m_ref, s_ref, didx_ref):
    g = pl.program_id(0)
    h = jnp.maximum(f_ref[...] @ w1_ref[...] + b1_ref[...][None, :], 0.0)
    h = jnp.maximum(h @ w2_ref[...] + b2_ref[...][None, :], 0.0)
    y = h @ w3_ref[...] + b3_ref[...][None, :]
    z27 = y @ k1m_ref[...]

    d = c_ref[0, :]
    hh = c_ref[1, :]
    ww = c_ref[2, :]
    vidx = g * VT + lax.broadcasted_iota(jnp.int32, (VT,), 0)
    in_range = vidx < NV
    for j in range(8):
        td, th, tw = (j >> 2) & 1, (j >> 1) & 1, j & 1
        od = (d >> 1) - td
        oh = (hh >> 1) - th
        ow = (ww >> 1) - tw
        kd = (d & 1) + 2 * td
        kh = (hh & 1) + 2 * th
        kw = (ww & 1) + 2 * tw
        valid = (kd < 3) & (kh < 3) & (kw < 3) & in_range
        kidx = kd * 9 + kh * 3 + kw
        acc = jnp.zeros((VT, 32), jnp.float32)
        for k in range(27):
            m = (kidx == k).astype(jnp.float32)[:, None]
            acc = acc + m * z27[:, k * 32:(k + 1) * 32]
        s_ref[j, :, :] = acc
        core = (od - 4) >> 4
        r = ((od - 4 - 16 * core) * L1HW + oh) * L1HW + ow
        for c in range(NC):
            didx_ref[c, j, :] = jnp.where(valid & (core == c), r, DUMMY)


def _prep(features, coordT, W1, b1, W2, b2, W3, b3, K1):
    k1m = jnp.transpose(K1.reshape(27, 32, 32), (1, 0, 2)).reshape(32, 27 * 32)
    f = jnp.pad(features.reshape(NV, 128), ((0, VP - NV), (0, 0)))
    s, didx = pl.pallas_call(
        _prep_body,
        grid=(NG,),
        in_specs=[
            pl.BlockSpec((VT, 128), lambda g: (g, 0)),
            pl.BlockSpec((3, VT), lambda g: (0, g)),
            pl.BlockSpec((128, 64), lambda g: (0, 0)),
            pl.BlockSpec((64,), lambda g: (0,)),
            pl.BlockSpec((64, 64), lambda g: (0, 0)),
            pl.BlockSpec((64,), lambda g: (0,)),
            pl.BlockSpec((64, 32), lambda g: (0, 0)),
            pl.BlockSpec((32,), lambda g: (0,)),
            pl.BlockSpec((32, 27 * 32), lambda g: (0, 0)),
        ],
        out_specs=[
            pl.BlockSpec((8, VT, 32), lambda g: (0, g, 0)),
            pl.BlockSpec((NC, 8, VT), lambda g: (0, 0, g)),
        ],
        out_shape=[
            jax.ShapeDtypeStruct((8, VP, 32), jnp.float32),
            jax.ShapeDtypeStruct((NC, 8, VP), jnp.int32),
        ],
    )(f, coordT, W1, b1, W2, b2, W3, b3, k1m)
    return s.reshape(NCONTRIB, 32), didx.reshape(NC, NS, 28, 128)


def _scatter_sc(s_rows, didx, zeros_hbm):
    mesh = plsc.VectorSubcoreMesh(core_axis_name="c", subcore_axis_name="s",
                                  num_cores=NC, num_subcores=NS)

    @functools.partial(
        pl.kernel,
        out_type=jax.ShapeDtypeStruct((L1DEP * PLANE, 32), jnp.float32),
        mesh=mesh,
        compiler_params=pltpu.CompilerParams(use_tc_tiling_on_sc=False),
        scratch_types=[
            pltpu.VMEM_SHARED((SHARD_ALL, 32), jnp.float32),
            pltpu.VMEM((128,), jnp.int32),
            pltpu.VMEM((128, 32), jnp.float32),
            pltpu.VMEM((CHUNK, 32), jnp.float32),
        ],
    )
    def body(s_hbm, didx_hbm, z_hbm, l1_hbm, shard, idx_v, buf_v, zbuf_v):
        c = lax.axis_index("c")
        s = lax.axis_index("s")
        wid = s * NC + c
        pltpu.sync_copy(z_hbm, zbuf_v)
        for i in range(PLANE // CHUNK):
            pltpu.sync_copy(zbuf_v, shard.at[pl.ds(s * PLANE + i * CHUNK, CHUNK)])
        @pl.when(s == 0)
        def _():
            pltpu.sync_copy(zbuf_v.at[pl.ds(0, 8)], shard.at[pl.ds(SHARD_ROWS, 8)])
        for i in range(7):
            k = wid + 32 * i
            row = jnp.where(k < 52, k * CHUNK, 36 * PLANE + (k - 52) * CHUNK)
            pltpu.sync_copy(zbuf_v, l1_hbm.at[pl.ds(row, CHUNK)])
        @pl.when(wid < 10)
        def _():
            k = wid + 224
            row = 36 * PLANE + (k - 52) * CHUNK
            pltpu.sync_copy(zbuf_v, l1_hbm.at[pl.ds(row, CHUNK)])
        plsc.subcore_barrier()
        for j in range(28):
            pltpu.sync_copy(didx_hbm.at[c, s, j], idx_v)
            pltpu.sync_copy(
                s_hbm.at[pl.ds(s * ROWS_PER_SUB + j * 128, 128)], buf_v)
            pltpu.sync_copy(buf_v, shard.at[idx_v], add=True)
        plsc.subcore_barrier()
        gbase = (4 + 16 * c + s) * PLANE
        for i in range(PLANE // CHUNK):
            pltpu.sync_copy(shard.at[pl.ds(s * PLANE + i * CHUNK, CHUNK)],
                            zbuf_v)
            def relu_row(r, _):
                v0 = jnp.maximum(zbuf_v[r, pl.ds(0, 16)], 0.0)
                zbuf_v[r, pl.ds(0, 16)] = v0
                v1 = jnp.maximum(zbuf_v[r, pl.ds(16, 16)], 0.0)
                zbuf_v[r, pl.ds(16, 16)] = v1
                return 0
            lax.fori_loop(0, CHUNK, relu_row, 0)
            pltpu.sync_copy(zbuf_v, l1_hbm.at[pl.ds(gbase + i * CHUNK, CHUNK)])

    return body(s_rows, didx, zeros_hbm)


def _conv_body(p0_ref, p1_ref, p2_ref, k_ref, o_ref, *, IN, OUT, OPAD):
    od = pl.program_id(0)
    HP = IN // 2 + IN % 2

    @pl.when(od < OUT)
    def _():
        acc = jnp.zeros((OUT * OUT, 32), jnp.float32)
        for kd, pref in ((0, p0_ref), (1, p1_ref), (2, p2_ref)):
            plane = pref[...].reshape(IN, IN, 32)
            pr = plane.reshape(HP, 2, HP, 2, 32)
            for kh in range(3):
                th, sh = kh >> 1, kh & 1
                for kw in range(3):
                    tw, sw = kw >> 1, kw & 1
                    term = pr[th:th + OUT, sh, tw:tw + OUT, sw, :]
                    acc = acc + term.reshape(OUT * OUT, 32) @ k_ref[kd, kh, kw]
        res = jnp.maximum(acc, 0.0).reshape(OUT, OUT, 32)
        res = jnp.pad(res, ((0, OPAD - OUT), (0, OPAD - OUT), (0, 0)))
        o_ref[...] = res.reshape(1, OPAD, OPAD, 32)

    @pl.when(od >= OUT)
    def _():
        o_ref[...] = jnp.zeros((1, OPAD, OPAD, 32), jnp.float32)


def _conv_level(vol, K, IN, OUT, OPAD, GRID):
    body = functools.partial(_conv_body, IN=IN, OUT=OUT, OPAD=OPAD)
    imap = [(lambda od, kd=kd: (jnp.minimum(2 * od + kd, IN - 1), 0, 0, 0))
            for kd in range(3)]
    return pl.pallas_call(
        body,
        grid=(GRID,),
        in_specs=[
            pl.BlockSpec((1, IN, IN, 32), imap[0]),
            pl.BlockSpec((1, IN, IN, 32), imap[1]),
            pl.BlockSpec((1, IN, IN, 32), imap[2]),
            pl.BlockSpec((3, 3, 3, 32, 32), lambda od: (0, 0, 0, 0, 0)),
        ],
        out_specs=pl.BlockSpec((1, OPAD, OPAD, 32), lambda od: (od, 0, 0, 0)),
        out_shape=jax.ShapeDtypeStruct((GRID, OPAD, OPAD, 32), jnp.float32),
    )(vol, vol, vol, K)


def _sample3d(vol, grid, dims):
    C = vol.shape[-1]
    D, H, W = dims[0], dims[1], dims[2]
    x = (grid[:, 0] + 1.0) * W * 0.5 - 0.5
    y = (grid[:, 1] + 1.0) * H * 0.5 - 0.5
    z = (grid[:, 2] + 1.0) * D * 0.5 - 0.5
    x0 = jnp.floor(x); y0 = jnp.floor(y); z0 = jnp.floor(z)
    wx = x - x0; wy = y - y0; wz = z - z0
    out = jnp.zeros((grid.shape[0], C), vol.dtype)
    for dz in (0, 1):
        for dy in (0, 1):
            for dx in (0, 1):
                zi = jnp.clip(z0 + dz, 0, D - 1).astype(jnp.int32)
                yi = jnp.clip(y0 + dy, 0, H - 1).astype(jnp.int32)
                xi = jnp.clip(x0 + dx, 0, W - 1).astype(jnp.int32)
                w = ((wz if dz else 1 - wz) * (wy if dy else 1 - wy) * (wx if dx else 1 - wx))[:, None]
                out = out + w * vol[zi, yi, xi]
    return out


def kernel(features, cnl_verts, canonical_pts, W1, b1, W2, b2, W3, b3, K1, K2, K3, K4):
    min_xyz = jnp.min(cnl_verts, axis=1) - 0.05
    max_xyz = jnp.max(cnl_verts, axis=1) + 0.05
    min_dhw = min_xyz[:, jnp.array([2, 1, 0])]
    max_dhw = max_xyz[:, jnp.array([2, 1, 0])]
    dhw = cnl_verts[..., jnp.array([2, 1, 0])]
    coord = jnp.round((dhw - min_dhw[:, None]) / VOXEL).astype(jnp.int32)
    out_sh = jnp.ceil((max_dhw - min_dhw) / VOXEL).astype(jnp.int32)
    out_sh = (out_sh | 31) + 1
    out_sh = jnp.max(out_sh, axis=0)

    coordT = jnp.pad(coord.reshape(NV, 3), ((0, VP - NV), (0, 0))).T
    s_rows, didx = _prep(features, coordT, W1, b1, W2, b2, W3, b3, K1)

    zeros_hbm = jnp.zeros((CHUNK, 32), jnp.float32)
    l1 = _scatter_sc(s_rows, didx, zeros_hbm).reshape(L1DEP, L1HW, L1HW, 32)

    l2 = _conv_level(l1, K2, 52, 24, 26, 26)
    l3 = _conv_level(l2, K3, 26, 12, 14, 14)
    l4 = _conv_level(l3, K4, 14, 6, 6, 6)

    q = canonical_pts[..., jnp.array([2, 1, 0])] - min_dhw[:, None]
    q = q / VOXEL
    q = q / jnp.asarray(out_sh, jnp.float32) * 2.0 - 1.0
    grid = q[..., jnp.array([2, 1, 0])][0]
    feats = []
    for v, dims in ((l1[:48, :48, :48], out_sh // 2),
                    (l2[:24, :24, :24], out_sh // 4),
                    (l3[:12, :12, :12], out_sh // 8),
                    (l4, out_sh // 16)):
        feats.append(_sample3d(v, grid, dims))
    return jnp.concatenate(feats, axis=-1)[None]


# speedup vs baseline: 27.5110x; 27.0941x over previous
import functools

import jax
import jax.numpy as jnp
from jax import lax
from jax.experimental import pallas as pl
from jax.experimental.pallas import tpu as pltpu
from jax.experimental.pallas import tpu_sc as plsc

VOXEL = 0.005
NC, NS = 2, 16
NTILES = NC * NS
NV = 6890
VT = 512
NG = 14
VP = NG * VT
NCONTRIB = 8 * VP
ROWS_PER_SUB = NCONTRIB // NS
L1DEP = 50
L1HW = 52
PLANE = L1HW * L1HW
CHUNK = 208
SHARD_PLANES = 16
SHARD_ROWS = SHARD_PLANES * PLANE
DUMMY = SHARD_ROWS
SHARD_ALL = SHARD_ROWS + 8


def _prep_body(f_ref, c_ref, w1_ref, b1_ref, w2_ref, b2_ref, w3_ref, b3_ref,
               k1m_ref, s_ref, didx_ref):
    g = pl.program_id(0)
    h = jnp.maximum(f_ref[...] @ w1_ref[...] + b1_ref[...][None, :], 0.0)
    h = jnp.maximum(h @ w2_ref[...] + b2_ref[...][None, :], 0.0)
    y = h @ w3_ref[...] + b3_ref[...][None, :]
    z27 = y @ k1m_ref[...]

    d = c_ref[0, :]
    hh = c_ref[1, :]
    ww = c_ref[2, :]
    vidx = g * VT + lax.broadcasted_iota(jnp.int32, (VT,), 0)
    in_range = vidx < NV
    for j in range(8):
        td, th, tw = (j >> 2) & 1, (j >> 1) & 1, j & 1
        od = (d >> 1) - td
        oh = (hh >> 1) - th
        ow = (ww >> 1) - tw
        kd = (d & 1) + 2 * td
        kh = (hh & 1) + 2 * th
        kw = (ww & 1) + 2 * tw
        valid = (kd < 3) & (kh < 3) & (kw < 3) & in_range
        kidx = kd * 9 + kh * 3 + kw
        acc = jnp.zeros((VT, 32), jnp.float32)
        for k in range(27):
            m = (kidx == k).astype(jnp.float32)[:, None]
            acc = acc + m * z27[:, k * 32:(k + 1) * 32]
        s_ref[j, :, :] = acc
        core = (od - 4) >> 4
        r = ((od - 4 - 16 * core) * L1HW + oh) * L1HW + ow
        for c in range(NC):
            didx_ref[c, j, :] = jnp.where(valid & (core == c), r, DUMMY)


def _prep(features, coordT, W1, b1, W2, b2, W3, b3, K1):
    k1m = jnp.transpose(K1.reshape(27, 32, 32), (1, 0, 2)).reshape(32, 27 * 32)
    f = jnp.pad(features.reshape(NV, 128), ((0, VP - NV), (0, 0)))
    s, didx = pl.pallas_call(
        _prep_body,
        grid=(NG,),
        in_specs=[
            pl.BlockSpec((VT, 128), lambda g: (g, 0)),
            pl.BlockSpec((3, VT), lambda g: (0, g)),
            pl.BlockSpec((128, 64), lambda g: (0, 0)),
            pl.BlockSpec((64,), lambda g: (0,)),
            pl.BlockSpec((64, 64), lambda g: (0, 0)),
            pl.BlockSpec((64,), lambda g: (0,)),
            pl.BlockSpec((64, 32), lambda g: (0, 0)),
            pl.BlockSpec((32,), lambda g: (0,)),
            pl.BlockSpec((32, 27 * 32), lambda g: (0, 0)),
        ],
        out_specs=[
            pl.BlockSpec((8, VT, 32), lambda g: (0, g, 0)),
            pl.BlockSpec((NC, 8, VT), lambda g: (0, 0, g)),
        ],
        out_shape=[
            jax.ShapeDtypeStruct((8, VP, 32), jnp.float32),
            jax.ShapeDtypeStruct((NC, 8, VP), jnp.int32),
        ],
    )(f, coordT, W1, b1, W2, b2, W3, b3, k1m)
    return s.reshape(NCONTRIB, 32), didx.reshape(NC, NS, 28, 128)


def _scatter_sc(s_rows, didx, zeros_hbm):
    mesh = plsc.VectorSubcoreMesh(core_axis_name="c", subcore_axis_name="s",
                                  num_cores=NC, num_subcores=NS)

    @functools.partial(
        pl.kernel,
        out_type=jax.ShapeDtypeStruct((L1DEP * PLANE, 32), jnp.float32),
        mesh=mesh,
        compiler_params=pltpu.CompilerParams(use_tc_tiling_on_sc=False),
        scratch_types=[
            pltpu.VMEM_SHARED((SHARD_ALL, 32), jnp.float32),
            pltpu.VMEM((128,), jnp.int32),
            pltpu.VMEM((128, 32), jnp.float32),
            pltpu.VMEM((CHUNK, 32), jnp.float32),
        ],
    )
    def body(s_hbm, didx_hbm, z_hbm, l1_hbm, shard, idx_v, buf_v, zbuf_v):
        c = lax.axis_index("c")
        s = lax.axis_index("s")
        wid = s * NC + c
        pltpu.sync_copy(z_hbm, zbuf_v)
        for i in range(PLANE // CHUNK):
            pltpu.sync_copy(zbuf_v, shard.at[pl.ds(s * PLANE + i * CHUNK, CHUNK)])
        @pl.when(s == 0)
        def _():
            pltpu.sync_copy(zbuf_v.at[pl.ds(0, 8)], shard.at[pl.ds(SHARD_ROWS, 8)])
        for i in range(7):
            k = wid + 32 * i
            row = jnp.where(k < 52, k * CHUNK, 36 * PLANE + (k - 52) * CHUNK)
            pltpu.sync_copy(zbuf_v, l1_hbm.at[pl.ds(row, CHUNK)])
        @pl.when(wid < 10)
        def _():
            k = wid + 224
            row = 36 * PLANE + (k - 52) * CHUNK
            pltpu.sync_copy(zbuf_v, l1_hbm.at[pl.ds(row, CHUNK)])
        plsc.subcore_barrier()
        for j in range(28):
            pltpu.sync_copy(didx_hbm.at[c, s, j], idx_v)
            pltpu.sync_copy(
                s_hbm.at[pl.ds(s * ROWS_PER_SUB + j * 128, 128)], buf_v)
            pltpu.sync_copy(buf_v, shard.at[idx_v], add=True)
        plsc.subcore_barrier()
        gbase = (4 + 16 * c + s) * PLANE
        for i in range(PLANE // CHUNK):
            pltpu.sync_copy(shard.at[pl.ds(s * PLANE + i * CHUNK, CHUNK)],
                            zbuf_v)
            def relu_row(r, _):
                v0 = jnp.maximum(zbuf_v[r, pl.ds(0, 16)], 0.0)
                zbuf_v[r, pl.ds(0, 16)] = v0
                v1 = jnp.maximum(zbuf_v[r, pl.ds(16, 16)], 0.0)
                zbuf_v[r, pl.ds(16, 16)] = v1
                return 0
            lax.fori_loop(0, CHUNK, relu_row, 0)
            pltpu.sync_copy(zbuf_v, l1_hbm.at[pl.ds(gbase + i * CHUNK, CHUNK)])

    return body(s_rows, didx, zeros_hbm)


def _conv_body(p0_ref, p1_ref, p2_ref, k_ref, o_ref, *, IN, OUT, OPAD):
    od = pl.program_id(0)
    HP = IN // 2 + IN % 2

    @pl.when(od < OUT)
    def _():
        acc = jnp.zeros((OUT * OUT, 32), jnp.float32)
        for kd, pref in ((0, p0_ref), (1, p1_ref), (2, p2_ref)):
            plane = pref[...].reshape(IN, IN, 32)
            pr = plane.reshape(HP, 2, HP, 2, 32)
            for kh in range(3):
                th, sh = kh >> 1, kh & 1
                for kw in range(3):
                    tw, sw = kw >> 1, kw & 1
                    term = pr[th:th + OUT, sh, tw:tw + OUT, sw, :]
                    acc = acc + term.reshape(OUT * OUT, 32) @ k_ref[kd, kh, kw]
        res = jnp.maximum(acc, 0.0).reshape(OUT, OUT, 32)
        res = jnp.pad(res, ((0, OPAD - OUT), (0, OPAD - OUT), (0, 0)))
        o_ref[...] = res.reshape(1, OPAD, OPAD, 32)

    @pl.when(od >= OUT)
    def _():
        o_ref[...] = jnp.zeros((1, OPAD, OPAD, 32), jnp.float32)


def _conv_level(vol, K, IN, OUT, OPAD, GRID):
    body = functools.partial(_conv_body, IN=IN, OUT=OUT, OPAD=OPAD)
    imap = [(lambda od, kd=kd: (jnp.minimum(2 * od + kd, IN - 1), 0, 0, 0))
            for kd in range(3)]
    return pl.pallas_call(
        body,
        grid=(GRID,),
        in_specs=[
            pl.BlockSpec((1, IN, IN, 32), imap[0]),
            pl.BlockSpec((1, IN, IN, 32), imap[1]),
            pl.BlockSpec((1, IN, IN, 32), imap[2]),
            pl.BlockSpec((3, 3, 3, 32, 32), lambda od: (0, 0, 0, 0, 0)),
        ],
        out_specs=pl.BlockSpec((1, OPAD, OPAD, 32), lambda od: (od, 0, 0, 0)),
        out_shape=jax.ShapeDtypeStruct((GRID, OPAD, OPAD, 32), jnp.float32),
    )(vol, vol, vol, K)


NP = 100000
NPP = 102400
PCHUNK = 128
NCHUNKS = NPP // (NTILES * PCHUNK)
LV_DIMS = ((50, 52, 52), (26, 26, 26), (14, 14, 14), (6, 6, 6))


def _ptprep_body(p_ref, sc_ref, idx_ref, w_ref):
    px = p_ref[0, :]
    py = p_ref[1, :]
    pz = p_ref[2, :]
    for l in range(4):
        crd = []
        for a, pa in enumerate((px, py, pz)):
            m = sc_ref[0, a]
            osh = sc_ref[0, 3 + a]
            u = (pa - m) / VOXEL
            g = u / osh * 2.0 - 1.0
            dim_l = osh * (1.0 / (1 << (l + 1)))
            crd.append(((g + 1.0) * dim_l * 0.5 - 0.5, dim_l))
        (xx, wd), (yy, hd), (zz, dd) = crd
        x0 = jnp.floor(xx); y0 = jnp.floor(yy); z0 = jnp.floor(zz)
        wx = xx - x0; wy = yy - y0; wz = zz - z0
        PD, PH, PW = LV_DIMS[l]
        for t in range(8):
            dz, dy, dx = (t >> 2) & 1, (t >> 1) & 1, t & 1
            zi = jnp.clip(z0 + dz, 0.0, dd - 1.0).astype(jnp.int32)
            yi = jnp.clip(y0 + dy, 0.0, hd - 1.0).astype(jnp.int32)
            xi = jnp.clip(x0 + dx, 0.0, wd - 1.0).astype(jnp.int32)
            w = ((wz if dz else 1.0 - wz) * (wy if dy else 1.0 - wy)
                 * (wx if dx else 1.0 - wx))
            flat = (zi * PH + yi) * PW + xi
            idx_ref[0, l, t, :] = flat
            w_ref[0, l, t, :] = w


def _ptprep(ptsT, scal):
    return pl.pallas_call(
        _ptprep_body,
        grid=(NTILES,),
        in_specs=[
            pl.BlockSpec((3, NCHUNKS * PCHUNK), lambda g: (0, g)),
            pl.BlockSpec((1, 8), lambda g: (0, 0)),
        ],
        out_specs=[
            pl.BlockSpec((1, 4, 8, NCHUNKS * PCHUNK), lambda g: (g, 0, 0, 0)),
            pl.BlockSpec((1, 4, 8, NCHUNKS * PCHUNK), lambda g: (g, 0, 0, 0)),
        ],
        out_shape=[
            jax.ShapeDtypeStruct((NTILES, 4, 8, NCHUNKS * PCHUNK), jnp.int32),
            jax.ShapeDtypeStruct((NTILES, 4, 8, NCHUNKS * PCHUNK), jnp.float32),
        ],
    )(ptsT, scal)


def _sample_sc(l1f, l2f, l3f, l4f, idx, w8):
    mesh = plsc.VectorSubcoreMesh(core_axis_name="c", subcore_axis_name="s",
                                  num_cores=NC, num_subcores=NS)

    @functools.partial(
        pl.kernel,
        out_type=jax.ShapeDtypeStruct((NPP, 128), jnp.float32),
        mesh=mesh,
        compiler_params=pltpu.CompilerParams(use_tc_tiling_on_sc=False),
        scratch_types=[
            pltpu.VMEM((8, NCHUNKS * PCHUNK), jnp.int32),
            pltpu.VMEM((8, NCHUNKS * PCHUNK), jnp.float32),
            pltpu.VMEM((8 * PCHUNK, 32), jnp.float32),
            pltpu.VMEM((PCHUNK, 32), jnp.float32),
            pltpu.SemaphoreType.DMA,
        ],
    )
    def body(v1, v2, v3, v4, idx_hbm, w_hbm, out_hbm,
             idx_v, w_v, rows_v, out_v, sem):
        c = lax.axis_index("c")
        s = lax.axis_index("s")
        wid = s * NC + c
        vols = (v1, v2, v3, v4)

        for l in range(4):
            for t in range(8):
                pltpu.sync_copy(idx_hbm.at[wid, l, t], idx_v.at[t])
                pltpu.sync_copy(w_hbm.at[wid, l, t], w_v.at[t])

            def do_chunk(ch, _, l=l):
                descs = [
                    pltpu.async_copy(
                        vols[l].at[idx_v.at[t, pl.ds(ch * PCHUNK, PCHUNK)]],
                        rows_v.at[pl.ds(t * PCHUNK, PCHUNK)], sem)
                    for t in range(8)
                ]
                for dsc in descs:
                    dsc.wait()

                def group(g, _):
                    pbase = g * 16
                    wvecs = [w_v[t, pl.ds(ch * PCHUNK + pbase, 16)]
                             for t in range(8)]
                    for pp in range(16):
                        p = pbase + pp
                        for half in range(2):
                            acc = jnp.zeros((16,), jnp.float32)
                            for t in range(8):
                                acc = acc + wvecs[t][pp] * rows_v[
                                    t * PCHUNK + p, pl.ds(half * 16, 16)]
                            out_v[p, pl.ds(half * 16, 16)] = acc
                    return 0
                lax.fori_loop(0, PCHUNK // 16, group, 0)
                base = (wid * NCHUNKS + ch) * PCHUNK
                pltpu.sync_copy(out_v,
                                out_hbm.at[pl.ds(base, PCHUNK),
                                           pl.ds(l * 32, 32)])
                return 0

            lax.fori_loop(0, NCHUNKS, do_chunk, 0)

    return body(l1f, l2f, l3f, l4f, idx, w8)


def _sample3d(vol, grid, dims):
    C = vol.shape[-1]
    D, H, W = dims[0], dims[1], dims[2]
    x = (grid[:, 0] + 1.0) * W * 0.5 - 0.5
    y = (grid[:, 1] + 1.0) * H * 0.5 - 0.5
    z = (grid[:, 2] + 1.0) * D * 0.5 - 0.5
    x0 = jnp.floor(x); y0 = jnp.floor(y); z0 = jnp.floor(z)
    wx = x - x0; wy = y - y0; wz = z - z0
    out = jnp.zeros((grid.shape[0], C), vol.dtype)
    for dz in (0, 1):
        for dy in (0, 1):
            for dx in (0, 1):
                zi = jnp.clip(z0 + dz, 0, D - 1).astype(jnp.int32)
                yi = jnp.clip(y0 + dy, 0, H - 1).astype(jnp.int32)
                xi = jnp.clip(x0 + dx, 0, W - 1).astype(jnp.int32)
                w = ((wz if dz else 1 - wz) * (wy if dy else 1 - wy) * (wx if dx else 1 - wx))[:, None]
                out = out + w * vol[zi, yi, xi]
    return out


def kernel(features, cnl_verts, canonical_pts, W1, b1, W2, b2, W3, b3, K1, K2, K3, K4):
    min_xyz = jnp.min(cnl_verts, axis=1) - 0.05
    max_xyz = jnp.max(cnl_verts, axis=1) + 0.05
    min_dhw = min_xyz[:, jnp.array([2, 1, 0])]
    max_dhw = max_xyz[:, jnp.array([2, 1, 0])]
    dhw = cnl_verts[..., jnp.array([2, 1, 0])]
    coord = jnp.round((dhw - min_dhw[:, None]) / VOXEL).astype(jnp.int32)
    out_sh = jnp.ceil((max_dhw - min_dhw) / VOXEL).astype(jnp.int32)
    out_sh = (out_sh | 31) + 1
    out_sh = jnp.max(out_sh, axis=0)

    coordT = jnp.pad(coord.reshape(NV, 3), ((0, VP - NV), (0, 0))).T
    s_rows, didx = _prep(features, coordT, W1, b1, W2, b2, W3, b3, K1)

    zeros_hbm = jnp.zeros((CHUNK, 32), jnp.float32)
    l1_rows = _scatter_sc(s_rows, didx, zeros_hbm)
    l1 = l1_rows.reshape(L1DEP, L1HW, L1HW, 32)

    l2 = _conv_level(l1, K2, 52, 24, 26, 26)
    l3 = _conv_level(l2, K3, 26, 12, 14, 14)
    l4 = _conv_level(l3, K4, 14, 6, 6, 6)

    ptsT = jnp.pad(canonical_pts.reshape(NP, 3), ((0, NPP - NP), (0, 0))).T
    osh_f = jnp.asarray(out_sh, jnp.float32)
    scal = jnp.concatenate([min_xyz[0], osh_f[jnp.array([2, 1, 0])],
                            jnp.zeros((2,), jnp.float32)]).reshape(1, 8)
    idx, w8 = _ptprep(ptsT, scal)
    out = _sample_sc(l1_rows, l2.reshape(-1, 32), l3.reshape(-1, 32),
                     l4.reshape(-1, 32), idx, w8)
    return out[:NP][None]


# speedup vs baseline: 28.9798x; 1.0534x over previous
import functools

import jax
import jax.numpy as jnp
from jax import lax
from jax.experimental import pallas as pl
from jax.experimental.pallas import tpu as pltpu
from jax.experimental.pallas import tpu_sc as plsc

VOXEL = 0.005
NC, NS = 2, 16
NTILES = NC * NS
NV = 6890
VT = 512
NG = 14
VP = NG * VT
NCONTRIB = 8 * VP
ROWS_PER_SUB = NCONTRIB // NS
L1DEP = 50
L1HW = 52
PLANE = L1HW * L1HW
CHUNK = 208
SHARD_PLANES = 16
SHARD_ROWS = SHARD_PLANES * PLANE
DUMMY = SHARD_ROWS
SHARD_ALL = SHARD_ROWS + 8


def _prep_body(f_ref, c_ref, w1_ref, b1_ref, w2_ref, b2_ref, w3_ref, b3_ref,
               k1m_ref, s_ref, didx_ref):
    g = pl.program_id(0)
    h = jnp.maximum(f_ref[...] @ w1_ref[...] + b1_ref[...][None, :], 0.0)
    h = jnp.maximum(h @ w2_ref[...] + b2_ref[...][None, :], 0.0)
    y = h @ w3_ref[...] + b3_ref[...][None, :]
    z27 = y @ k1m_ref[...]

    d = c_ref[0, :]
    hh = c_ref[1, :]
    ww = c_ref[2, :]
    vidx = g * VT + lax.broadcasted_iota(jnp.int32, (VT,), 0)
    in_range = vidx < NV
    for j in range(8):
        td, th, tw = (j >> 2) & 1, (j >> 1) & 1, j & 1
        od = (d >> 1) - td
        oh = (hh >> 1) - th
        ow = (ww >> 1) - tw
        kd = (d & 1) + 2 * td
        kh = (hh & 1) + 2 * th
        kw = (ww & 1) + 2 * tw
        valid = (kd < 3) & (kh < 3) & (kw < 3) & in_range
        kidx = kd * 9 + kh * 3 + kw
        acc = jnp.zeros((VT, 32), jnp.float32)
        for k in range(27):
            m = (kidx == k).astype(jnp.float32)[:, None]
            acc = acc + m * z27[:, k * 32:(k + 1) * 32]
        s_ref[j, :, :] = acc
        core = (od - 4) >> 4
        r = ((od - 4 - 16 * core) * L1HW + oh) * L1HW + ow
        for c in range(NC):
            didx_ref[c, j, :] = jnp.where(valid & (core == c), r, DUMMY)


def _prep(features, coordT, W1, b1, W2, b2, W3, b3, K1):
    k1m = jnp.transpose(K1.reshape(27, 32, 32), (1, 0, 2)).reshape(32, 27 * 32)
    f = jnp.pad(features.reshape(NV, 128), ((0, VP - NV), (0, 0)))
    s, didx = pl.pallas_call(
        _prep_body,
        grid=(NG,),
        in_specs=[
            pl.BlockSpec((VT, 128), lambda g: (g, 0)),
            pl.BlockSpec((3, VT), lambda g: (0, g)),
            pl.BlockSpec((128, 64), lambda g: (0, 0)),
            pl.BlockSpec((64,), lambda g: (0,)),
            pl.BlockSpec((64, 64), lambda g: (0, 0)),
            pl.BlockSpec((64,), lambda g: (0,)),
            pl.BlockSpec((64, 32), lambda g: (0, 0)),
            pl.BlockSpec((32,), lambda g: (0,)),
            pl.BlockSpec((32, 27 * 32), lambda g: (0, 0)),
        ],
        out_specs=[
            pl.BlockSpec((8, VT, 32), lambda g: (0, g, 0)),
            pl.BlockSpec((NC, 8, VT), lambda g: (0, 0, g)),
        ],
        out_shape=[
            jax.ShapeDtypeStruct((8, VP, 32), jnp.float32),
            jax.ShapeDtypeStruct((NC, 8, VP), jnp.int32),
        ],
    )(f, coordT, W1, b1, W2, b2, W3, b3, k1m)
    return s.reshape(NCONTRIB, 32), didx.reshape(NC, NS, 28, 128)


def _scatter_sc(s_rows, didx, zeros_hbm):
    mesh = plsc.VectorSubcoreMesh(core_axis_name="c", subcore_axis_name="s",
                                  num_cores=NC, num_subcores=NS)

    @functools.partial(
        pl.kernel,
        out_type=jax.ShapeDtypeStruct((L1DEP * PLANE, 32), jnp.float32),
        mesh=mesh,
        compiler_params=pltpu.CompilerParams(use_tc_tiling_on_sc=False),
        scratch_types=[
            pltpu.VMEM_SHARED((SHARD_ALL, 32), jnp.float32),
            pltpu.VMEM((128,), jnp.int32),
            pltpu.VMEM((128, 32), jnp.float32),
            pltpu.VMEM((CHUNK, 32), jnp.float32),
        ],
    )
    def body(s_hbm, didx_hbm, z_hbm, l1_hbm, shard, idx_v, buf_v, zbuf_v):
        c = lax.axis_index("c")
        s = lax.axis_index("s")
        wid = s * NC + c
        pltpu.sync_copy(z_hbm, zbuf_v)
        for i in range(PLANE // CHUNK):
            pltpu.sync_copy(zbuf_v, shard.at[pl.ds(s * PLANE + i * CHUNK, CHUNK)])
        @pl.when(s == 0)
        def _():
            pltpu.sync_copy(zbuf_v.at[pl.ds(0, 8)], shard.at[pl.ds(SHARD_ROWS, 8)])
        for i in range(7):
            k = wid + 32 * i
            row = jnp.where(k < 52, k * CHUNK, 36 * PLANE + (k - 52) * CHUNK)
            pltpu.sync_copy(zbuf_v, l1_hbm.at[pl.ds(row, CHUNK)])
        @pl.when(wid < 10)
        def _():
            k = wid + 224
            row = 36 * PLANE + (k - 52) * CHUNK
            pltpu.sync_copy(zbuf_v, l1_hbm.at[pl.ds(row, CHUNK)])
        plsc.subcore_barrier()
        for j in range(28):
            pltpu.sync_copy(didx_hbm.at[c, s, j], idx_v)
            pltpu.sync_copy(
                s_hbm.at[pl.ds(s * ROWS_PER_SUB + j * 128, 128)], buf_v)
            pltpu.sync_copy(buf_v, shard.at[idx_v], add=True)
        plsc.subcore_barrier()
        gbase = (4 + 16 * c + s) * PLANE
        for i in range(PLANE // CHUNK):
            pltpu.sync_copy(shard.at[pl.ds(s * PLANE + i * CHUNK, CHUNK)],
                            zbuf_v)
            def relu_row(r, _):
                v0 = jnp.maximum(zbuf_v[r, pl.ds(0, 16)], 0.0)
                zbuf_v[r, pl.ds(0, 16)] = v0
                v1 = jnp.maximum(zbuf_v[r, pl.ds(16, 16)], 0.0)
                zbuf_v[r, pl.ds(16, 16)] = v1
                return 0
            lax.fori_loop(0, CHUNK, relu_row, 0)
            pltpu.sync_copy(zbuf_v, l1_hbm.at[pl.ds(gbase + i * CHUNK, CHUNK)])

    return body(s_rows, didx, zeros_hbm)


def _conv_body(p0_ref, p1_ref, p2_ref, k_ref, o_ref, *, IN, OUT, OPAD):
    od = pl.program_id(0)
    HP = IN // 2 + IN % 2

    @pl.when(od < OUT)
    def _():
        acc = jnp.zeros((OUT * OUT, 32), jnp.float32)
        for kd, pref in ((0, p0_ref), (1, p1_ref), (2, p2_ref)):
            plane = pref[...].reshape(IN, IN, 32)
            pr = plane.reshape(HP, 2, HP, 2, 32)
            for kh in range(3):
                th, sh = kh >> 1, kh & 1
                for kw in range(3):
                    tw, sw = kw >> 1, kw & 1
                    term = pr[th:th + OUT, sh, tw:tw + OUT, sw, :]
                    acc = acc + term.reshape(OUT * OUT, 32) @ k_ref[kd, kh, kw]
        res = jnp.maximum(acc, 0.0).reshape(OUT, OUT, 32)
        res = jnp.pad(res, ((0, OPAD - OUT), (0, OPAD - OUT), (0, 0)))
        o_ref[...] = res.reshape(1, OPAD, OPAD, 32)

    @pl.when(od >= OUT)
    def _():
        o_ref[...] = jnp.zeros((1, OPAD, OPAD, 32), jnp.float32)


def _conv_level(vol, K, IN, OUT, OPAD, GRID):
    body = functools.partial(_conv_body, IN=IN, OUT=OUT, OPAD=OPAD)
    imap = [(lambda od, kd=kd: (jnp.minimum(2 * od + kd, IN - 1), 0, 0, 0))
            for kd in range(3)]
    return pl.pallas_call(
        body,
        grid=(GRID,),
        in_specs=[
            pl.BlockSpec((1, IN, IN, 32), imap[0]),
            pl.BlockSpec((1, IN, IN, 32), imap[1]),
            pl.BlockSpec((1, IN, IN, 32), imap[2]),
            pl.BlockSpec((3, 3, 3, 32, 32), lambda od: (0, 0, 0, 0, 0)),
        ],
        out_specs=pl.BlockSpec((1, OPAD, OPAD, 32), lambda od: (od, 0, 0, 0)),
        out_shape=jax.ShapeDtypeStruct((GRID, OPAD, OPAD, 32), jnp.float32),
    )(vol, vol, vol, K)


NP = 100000
NPP = 102400
PCHUNK = 128
NCHUNKS = NPP // (NTILES * PCHUNK)
LV_DIMS = ((50, 52, 52), (26, 26, 26), (14, 14, 14), (6, 6, 6))


def _ptprep_body(p_ref, sc_ref, idx_ref, w_ref):
    px = p_ref[0, :]
    py = p_ref[1, :]
    pz = p_ref[2, :]
    for l in range(4):
        crd = []
        for a, pa in enumerate((px, py, pz)):
            m = sc_ref[0, a]
            osh = sc_ref[0, 3 + a]
            u = (pa - m) / VOXEL
            g = u / osh * 2.0 - 1.0
            dim_l = osh * (1.0 / (1 << (l + 1)))
            crd.append(((g + 1.0) * dim_l * 0.5 - 0.5, dim_l))
        (xx, wd), (yy, hd), (zz, dd) = crd
        x0 = jnp.floor(xx); y0 = jnp.floor(yy); z0 = jnp.floor(zz)
        wx = xx - x0; wy = yy - y0; wz = zz - z0
        PD, PH, PW = LV_DIMS[l]
        for t in range(8):
            dz, dy, dx = (t >> 2) & 1, (t >> 1) & 1, t & 1
            zi = jnp.clip(z0 + dz, 0.0, dd - 1.0).astype(jnp.int32)
            yi = jnp.clip(y0 + dy, 0.0, hd - 1.0).astype(jnp.int32)
            xi = jnp.clip(x0 + dx, 0.0, wd - 1.0).astype(jnp.int32)
            w = ((wz if dz else 1.0 - wz) * (wy if dy else 1.0 - wy)
                 * (wx if dx else 1.0 - wx))
            flat = (zi * PH + yi) * PW + xi
            idx_ref[0, l, t, :] = flat
            w_ref[0, l, t, :] = w


def _ptprep(ptsT, scal):
    return pl.pallas_call(
        _ptprep_body,
        grid=(NTILES,),
        in_specs=[
            pl.BlockSpec((3, NCHUNKS * PCHUNK), lambda g: (0, g)),
            pl.BlockSpec((1, 8), lambda g: (0, 0)),
        ],
        out_specs=[
            pl.BlockSpec((1, 4, 8, NCHUNKS * PCHUNK), lambda g: (g, 0, 0, 0)),
            pl.BlockSpec((1, 4, 8, NCHUNKS * PCHUNK), lambda g: (g, 0, 0, 0)),
        ],
        out_shape=[
            jax.ShapeDtypeStruct((NTILES, 4, 8, NCHUNKS * PCHUNK), jnp.int32),
            jax.ShapeDtypeStruct((NTILES, 4, 8, NCHUNKS * PCHUNK), jnp.float32),
        ],
    )(ptsT, scal)


def _sample_sc(l1f, l2f, l3f, l4f, idx, w8):
    mesh = plsc.VectorSubcoreMesh(core_axis_name="c", subcore_axis_name="s",
                                  num_cores=NC, num_subcores=NS)

    @functools.partial(
        pl.kernel,
        out_type=jax.ShapeDtypeStruct((NPP, 128), jnp.float32),
        mesh=mesh,
        compiler_params=pltpu.CompilerParams(use_tc_tiling_on_sc=False),
        scratch_types=[
            pltpu.VMEM((8, NCHUNKS * PCHUNK), jnp.int32),
            pltpu.VMEM((8, NCHUNKS * PCHUNK), jnp.float32),
            pltpu.VMEM((2, 8 * PCHUNK, 32), jnp.float32),
            pltpu.VMEM((2, PCHUNK, 32), jnp.float32),
            pltpu.SemaphoreType.DMA,
            pltpu.SemaphoreType.DMA,
            pltpu.SemaphoreType.DMA,
            pltpu.SemaphoreType.DMA,
        ],
    )
    def body(v1, v2, v3, v4, idx_hbm, w_hbm, out_hbm,
             idx_v, w_v, rows_v, out_v, semA, semB, semOA, semOB):
        c = lax.axis_index("c")
        s = lax.axis_index("s")
        wid = s * NC + c
        vols = (v1, v2, v3, v4)
        sems = (semA, semB)
        osems = (semOA, semOB)

        for l in range(4):
            for t in range(8):
                pltpu.sync_copy(idx_hbm.at[wid, l, t], idx_v.at[t])
                pltpu.sync_copy(w_hbm.at[wid, l, t], w_v.at[t])

            def fire(ch, b, l=l):
                for t in range(8):
                    pltpu.async_copy(
                        vols[l].at[idx_v.at[t, pl.ds(ch * PCHUNK, PCHUNK)]],
                        rows_v.at[b, pl.ds(t * PCHUNK, PCHUNK)], sems[b])

            def drain(ch, b, l=l):
                for t in range(8):
                    pltpu.make_async_copy(
                        vols[l].at[idx_v.at[t, pl.ds(ch * PCHUNK, PCHUNK)]],
                        rows_v.at[b, pl.ds(t * PCHUNK, PCHUNK)],
                        sems[b]).wait()

            def compute(ch, b, l=l, first=False):
                def group(g, _):
                    pbase = g * 16
                    wvecs = [w_v[t, pl.ds(ch * PCHUNK + pbase, 16)]
                             for t in range(8)]
                    for pp in range(16):
                        p = pbase + pp
                        for half in range(2):
                            acc = jnp.zeros((16,), jnp.float32)
                            for t in range(8):
                                acc = acc + wvecs[t][pp] * rows_v[
                                    b, t * PCHUNK + p, pl.ds(half * 16, 16)]
                            out_v[b, p, pl.ds(half * 16, 16)] = acc
                    return 0
                base = (wid * NCHUNKS + ch) * PCHUNK
                dst = out_hbm.at[pl.ds(base, PCHUNK), pl.ds(l * 32, 32)]
                if not first:
                    pltpu.make_async_copy(out_v.at[b], dst, osems[b]).wait()
                lax.fori_loop(0, PCHUNK // 16, group, 0)
                pltpu.async_copy(out_v.at[b], dst, osems[b])

            fire(0, 0)
            fire(1, 1)
            drain(0, 0)
            compute(0, 0, first=True)
            fire(2, 0)
            drain(1, 1)
            compute(1, 1, first=True)

            def pair(ii, _, l=l):
                ch0 = 2 * ii
                fire(ch0 + 1, 1)
                drain(ch0, 0)
                compute(ch0, 0)
                fire(ch0 + 2, 0)
                drain(ch0 + 1, 1)
                compute(ch0 + 1, 1)
                return 0

            lax.fori_loop(1, (NCHUNKS - 1) // 2, pair, 0)
            drain(NCHUNKS - 1, 0)
            compute(NCHUNKS - 1, 0)
            for b in range(2):
                dst = out_hbm.at[pl.ds(wid * NCHUNKS * PCHUNK, PCHUNK),
                                 pl.ds(l * 32, 32)]
                pltpu.make_async_copy(out_v.at[b], dst, osems[b]).wait()

    return body(l1f, l2f, l3f, l4f, idx, w8)


def _sample3d(vol, grid, dims):
    C = vol.shape[-1]
    D, H, W = dims[0], dims[1], dims[2]
    x = (grid[:, 0] + 1.0) * W * 0.5 - 0.5
    y = (grid[:, 1] + 1.0) * H * 0.5 - 0.5
    z = (grid[:, 2] + 1.0) * D * 0.5 - 0.5
    x0 = jnp.floor(x); y0 = jnp.floor(y); z0 = jnp.floor(z)
    wx = x - x0; wy = y - y0; wz = z - z0
    out = jnp.zeros((grid.shape[0], C), vol.dtype)
    for dz in (0, 1):
        for dy in (0, 1):
            for dx in (0, 1):
                zi = jnp.clip(z0 + dz, 0, D - 1).astype(jnp.int32)
                yi = jnp.clip(y0 + dy, 0, H - 1).astype(jnp.int32)
                xi = jnp.clip(x0 + dx, 0, W - 1).astype(jnp.int32)
                w = ((wz if dz else 1 - wz) * (wy if dy else 1 - wy) * (wx if dx else 1 - wx))[:, None]
                out = out + w * vol[zi, yi, xi]
    return out


def kernel(features, cnl_verts, canonical_pts, W1, b1, W2, b2, W3, b3, K1, K2, K3, K4):
    min_xyz = jnp.min(cnl_verts, axis=1) - 0.05
    max_xyz = jnp.max(cnl_verts, axis=1) + 0.05
    min_dhw = min_xyz[:, jnp.array([2, 1, 0])]
    max_dhw = max_xyz[:, jnp.array([2, 1, 0])]
    dhw = cnl_verts[..., jnp.array([2, 1, 0])]
    coord = jnp.round((dhw - min_dhw[:, None]) / VOXEL).astype(jnp.int32)
    out_sh = jnp.ceil((max_dhw - min_dhw) / VOXEL).astype(jnp.int32)
    out_sh = (out_sh | 31) + 1
    out_sh = jnp.max(out_sh, axis=0)

    coordT = jnp.pad(coord.reshape(NV, 3), ((0, VP - NV), (0, 0))).T
    s_rows, didx = _prep(features, coordT, W1, b1, W2, b2, W3, b3, K1)

    zeros_hbm = jnp.zeros((CHUNK, 32), jnp.float32)
    l1_rows = _scatter_sc(s_rows, didx, zeros_hbm)
    l1 = l1_rows.reshape(L1DEP, L1HW, L1HW, 32)

    l2 = _conv_level(l1, K2, 52, 24, 26, 26)
    l3 = _conv_level(l2, K3, 26, 12, 14, 14)
    l4 = _conv_level(l3, K4, 14, 6, 6, 6)

    ptsT = jnp.pad(canonical_pts.reshape(NP, 3), ((0, NPP - NP), (0, 0))).T
    osh_f = jnp.asarray(out_sh, jnp.float32)
    scal = jnp.concatenate([min_xyz[0], osh_f[jnp.array([2, 1, 0])],
                            jnp.zeros((2,), jnp.float32)]).reshape(1, 8)
    idx, w8 = _ptprep(ptsT, scal)
    out = _sample_sc(l1_rows, l2.reshape(-1, 32), l3.reshape(-1, 32),
                     l4.reshape(-1, 32), idx, w8)
    return out[:NP][None]


# speedup vs baseline: 30.2523x; 1.0439x over previous
import functools

import jax
import jax.numpy as jnp
from jax import lax
from jax.experimental import pallas as pl
from jax.experimental.pallas import tpu as pltpu
from jax.experimental.pallas import tpu_sc as plsc

VOXEL = 0.005
NC, NS = 2, 16
NTILES = NC * NS
NV = 6890
VT = 512
NG = 14
VP = NG * VT
NCONTRIB = 8 * VP
ROWS_PER_SUB = NCONTRIB // NS
L1DEP = 50
L1HW = 52
PLANE = L1HW * L1HW
CHUNK = 208
SHARD_PLANES = 16
SHARD_ROWS = SHARD_PLANES * PLANE
DUMMY = SHARD_ROWS
SHARD_ALL = SHARD_ROWS + 8


def _prep_body(f_ref, c_ref, w1_ref, b1_ref, w2_ref, b2_ref, w3_ref, b3_ref,
               k1m_ref, z_ref, didx_ref, sidx_ref):
    g = pl.program_id(0)
    h = jnp.maximum(f_ref[...] @ w1_ref[...] + b1_ref[...][None, :], 0.0)
    h = jnp.maximum(h @ w2_ref[...] + b2_ref[...][None, :], 0.0)
    y = h @ w3_ref[...] + b3_ref[...][None, :]
    z_ref[...] = y @ k1m_ref[...]

    d = c_ref[0, :]
    hh = c_ref[1, :]
    ww = c_ref[2, :]
    vidx = g * VT + lax.broadcasted_iota(jnp.int32, (VT,), 0)
    in_range = vidx < NV
    for j in range(8):
        td, th, tw = (j >> 2) & 1, (j >> 1) & 1, j & 1
        od = (d >> 1) - td
        oh = (hh >> 1) - th
        ow = (ww >> 1) - tw
        kd = (d & 1) + 2 * td
        kh = (hh & 1) + 2 * th
        kw = (ww & 1) + 2 * tw
        valid = (kd < 3) & (kh < 3) & (kw < 3) & in_range
        kidx = kd * 9 + kh * 3 + kw
        sidx_ref[j, :] = vidx * 27 + jnp.minimum(kidx, 26)
        core = (od - 4) >> 4
        r = ((od - 4 - 16 * core) * L1HW + oh) * L1HW + ow
        for c in range(NC):
            didx_ref[c, j, :] = jnp.where(valid & (core == c), r, DUMMY)


def _prep(features, coordT, W1, b1, W2, b2, W3, b3, K1):
    k1m = jnp.transpose(K1.reshape(27, 32, 32), (1, 0, 2)).reshape(32, 27 * 32)
    f = jnp.pad(features.reshape(NV, 128), ((0, VP - NV), (0, 0)))
    s, didx, sidx = pl.pallas_call(
        _prep_body,
        grid=(NG,),
        in_specs=[
            pl.BlockSpec((VT, 128), lambda g: (g, 0)),
            pl.BlockSpec((3, VT), lambda g: (0, g)),
            pl.BlockSpec((128, 64), lambda g: (0, 0)),
            pl.BlockSpec((64,), lambda g: (0,)),
            pl.BlockSpec((64, 64), lambda g: (0, 0)),
            pl.BlockSpec((64,), lambda g: (0,)),
            pl.BlockSpec((64, 32), lambda g: (0, 0)),
            pl.BlockSpec((32,), lambda g: (0,)),
            pl.BlockSpec((32, 27 * 32), lambda g: (0, 0)),
        ],
        out_specs=[
            pl.BlockSpec((VT, 27 * 32), lambda g: (g, 0)),
            pl.BlockSpec((NC, 8, VT), lambda g: (0, 0, g)),
            pl.BlockSpec((8, VT), lambda g: (0, g)),
        ],
        out_shape=[
            jax.ShapeDtypeStruct((VP, 27 * 32), jnp.float32),
            jax.ShapeDtypeStruct((NC, 8, VP), jnp.int32),
            jax.ShapeDtypeStruct((8, VP), jnp.int32),
        ],
    )(f, coordT, W1, b1, W2, b2, W3, b3, k1m)
    return (s.reshape(VP * 27, 32), didx.reshape(NC, NS, 28, 128),
            sidx.reshape(NS, 28, 128))


def _scatter_sc(s_rows, didx, sidx, zeros_hbm):
    mesh = plsc.VectorSubcoreMesh(core_axis_name="c", subcore_axis_name="s",
                                  num_cores=NC, num_subcores=NS)

    @functools.partial(
        pl.kernel,
        out_type=jax.ShapeDtypeStruct((L1DEP * PLANE, 32), jnp.float32),
        mesh=mesh,
        compiler_params=pltpu.CompilerParams(use_tc_tiling_on_sc=False),
        scratch_types=[
            pltpu.VMEM_SHARED((SHARD_ALL, 32), jnp.float32),
            pltpu.VMEM((128,), jnp.int32),
            pltpu.VMEM((128,), jnp.int32),
            pltpu.VMEM((128, 32), jnp.float32),
            pltpu.VMEM((CHUNK, 32), jnp.float32),
            pltpu.SemaphoreType.DMA,
        ],
    )
    def body(s_hbm, didx_hbm, sidx_hbm, z_hbm, l1_hbm, shard, idx_v, sidx_v,
             buf_v, zbuf_v, gsem):
        c = lax.axis_index("c")
        s = lax.axis_index("s")
        wid = s * NC + c
        pltpu.sync_copy(z_hbm, zbuf_v)
        for i in range(PLANE // CHUNK):
            pltpu.sync_copy(zbuf_v, shard.at[pl.ds(s * PLANE + i * CHUNK, CHUNK)])
        @pl.when(s == 0)
        def _():
            pltpu.sync_copy(zbuf_v.at[pl.ds(0, 8)], shard.at[pl.ds(SHARD_ROWS, 8)])
        for i in range(7):
            k = wid + 32 * i
            row = jnp.where(k < 52, k * CHUNK, 36 * PLANE + (k - 52) * CHUNK)
            pltpu.sync_copy(zbuf_v, l1_hbm.at[pl.ds(row, CHUNK)])
        @pl.when(wid < 10)
        def _():
            k = wid + 224
            row = 36 * PLANE + (k - 52) * CHUNK
            pltpu.sync_copy(zbuf_v, l1_hbm.at[pl.ds(row, CHUNK)])
        plsc.subcore_barrier()
        for j in range(28):
            pltpu.sync_copy(didx_hbm.at[c, s, j], idx_v)
            pltpu.sync_copy(sidx_hbm.at[s, j], sidx_v)
            pltpu.async_copy(s_hbm.at[sidx_v], buf_v, gsem).wait()
            pltpu.sync_copy(buf_v, shard.at[idx_v], add=True)
        plsc.subcore_barrier()
        gbase = (4 + 16 * c + s) * PLANE
        for i in range(PLANE // CHUNK):
            pltpu.sync_copy(shard.at[pl.ds(s * PLANE + i * CHUNK, CHUNK)],
                            zbuf_v)
            def relu_row(r, _):
                v0 = jnp.maximum(zbuf_v[r, pl.ds(0, 16)], 0.0)
                zbuf_v[r, pl.ds(0, 16)] = v0
                v1 = jnp.maximum(zbuf_v[r, pl.ds(16, 16)], 0.0)
                zbuf_v[r, pl.ds(16, 16)] = v1
                return 0
            lax.fori_loop(0, CHUNK, relu_row, 0)
            pltpu.sync_copy(zbuf_v, l1_hbm.at[pl.ds(gbase + i * CHUNK, CHUNK)])

    return body(s_rows, didx, sidx, zeros_hbm)


def _conv_body(p0_ref, p1_ref, p2_ref, k_ref, o_ref, *, IN, OUT, OPAD):
    od = pl.program_id(0)
    HP = IN // 2 + IN % 2

    @pl.when(od < OUT)
    def _():
        acc = jnp.zeros((OUT * OUT, 32), jnp.float32)
        for kd, pref in ((0, p0_ref), (1, p1_ref), (2, p2_ref)):
            plane = pref[...].reshape(IN, IN, 32)
            pr = plane.reshape(HP, 2, HP, 2, 32)
            for kh in range(3):
                th, sh = kh >> 1, kh & 1
                for kw in range(3):
                    tw, sw = kw >> 1, kw & 1
                    term = pr[th:th + OUT, sh, tw:tw + OUT, sw, :]
                    acc = acc + term.reshape(OUT * OUT, 32) @ k_ref[kd, kh, kw]
        res = jnp.maximum(acc, 0.0).reshape(OUT, OUT, 32)
        res = jnp.pad(res, ((0, OPAD - OUT), (0, OPAD - OUT), (0, 0)))
        o_ref[...] = res.reshape(1, OPAD, OPAD, 32)

    @pl.when(od >= OUT)
    def _():
        o_ref[...] = jnp.zeros((1, OPAD, OPAD, 32), jnp.float32)


def _conv_level(vol, K, IN, OUT, OPAD, GRID):
    body = functools.partial(_conv_body, IN=IN, OUT=OUT, OPAD=OPAD)
    imap = [(lambda od, kd=kd: (jnp.minimum(2 * od + kd, IN - 1), 0, 0, 0))
            for kd in range(3)]
    return pl.pallas_call(
        body,
        grid=(GRID,),
        in_specs=[
            pl.BlockSpec((1, IN, IN, 32), imap[0]),
            pl.BlockSpec((1, IN, IN, 32), imap[1]),
            pl.BlockSpec((1, IN, IN, 32), imap[2]),
            pl.BlockSpec((3, 3, 3, 32, 32), lambda od: (0, 0, 0, 0, 0)),
        ],
        out_specs=pl.BlockSpec((1, OPAD, OPAD, 32), lambda od: (od, 0, 0, 0)),
        out_shape=jax.ShapeDtypeStruct((GRID, OPAD, OPAD, 32), jnp.float32),
    )(vol, vol, vol, K)


NP = 100000
NPP = 102400
PCHUNK = 128
NCHUNKS = NPP // (NTILES * PCHUNK)
LV_DIMS = ((50, 52, 52), (26, 26, 26), (14, 14, 14), (6, 6, 6))


def _ptprep_body(p_ref, sc_ref, idx_ref, w_ref):
    px = p_ref[0, :]
    py = p_ref[1, :]
    pz = p_ref[2, :]
    for l in range(4):
        crd = []
        for a, pa in enumerate((px, py, pz)):
            m = sc_ref[0, a]
            osh = sc_ref[0, 3 + a]
            u = (pa - m) / VOXEL
            g = u / osh * 2.0 - 1.0
            dim_l = osh * (1.0 / (1 << (l + 1)))
            crd.append(((g + 1.0) * dim_l * 0.5 - 0.5, dim_l))
        (xx, wd), (yy, hd), (zz, dd) = crd
        x0 = jnp.floor(xx); y0 = jnp.floor(yy); z0 = jnp.floor(zz)
        wx = xx - x0; wy = yy - y0; wz = zz - z0
        PD, PH, PW = LV_DIMS[l]
        for t in range(8):
            dz, dy, dx = (t >> 2) & 1, (t >> 1) & 1, t & 1
            zi = jnp.clip(z0 + dz, 0.0, dd - 1.0).astype(jnp.int32)
            yi = jnp.clip(y0 + dy, 0.0, hd - 1.0).astype(jnp.int32)
            xi = jnp.clip(x0 + dx, 0.0, wd - 1.0).astype(jnp.int32)
            w = ((wz if dz else 1.0 - wz) * (wy if dy else 1.0 - wy)
                 * (wx if dx else 1.0 - wx))
            idx_ref[0, l, t, :] = (zi * PH + yi) * PW + xi
            w_ref[0, l, t, :] = w


def _ptprep(ptsT, scal):
    return pl.pallas_call(
        _ptprep_body,
        grid=(NTILES,),
        in_specs=[
            pl.BlockSpec((3, NCHUNKS * PCHUNK), lambda g: (0, g)),
            pl.BlockSpec((1, 8), lambda g: (0, 0)),
        ],
        out_specs=[
            pl.BlockSpec((1, 4, 8, NCHUNKS * PCHUNK), lambda g: (g, 0, 0, 0)),
            pl.BlockSpec((1, 4, 8, NCHUNKS * PCHUNK), lambda g: (g, 0, 0, 0)),
        ],
        out_shape=[
            jax.ShapeDtypeStruct((NTILES, 4, 8, NCHUNKS * PCHUNK), jnp.int32),
            jax.ShapeDtypeStruct((NTILES, 4, 8, NCHUNKS * PCHUNK), jnp.float32),
        ],
    )(ptsT, scal)


def _sample_sc(l1f, l2f, l3f, l4f, idx, w8):
    mesh = plsc.VectorSubcoreMesh(core_axis_name="c", subcore_axis_name="s",
                                  num_cores=NC, num_subcores=NS)

    @functools.partial(
        pl.kernel,
        out_type=jax.ShapeDtypeStruct((NPP, 128), jnp.float32),
        mesh=mesh,
        compiler_params=pltpu.CompilerParams(use_tc_tiling_on_sc=False),
        scratch_types=[
            pltpu.VMEM((NCHUNKS * PCHUNK * 8,), jnp.int32),
            pltpu.VMEM((NCHUNKS * PCHUNK // 2, 16), jnp.float32),
            pltpu.VMEM((2, 8 * PCHUNK, 32), jnp.float32),
            pltpu.VMEM((2, PCHUNK, 32), jnp.float32),
            pltpu.SemaphoreType.DMA,
            pltpu.SemaphoreType.DMA,
            pltpu.SemaphoreType.DMA,
            pltpu.SemaphoreType.DMA,
        ],
    )
    def body(v1, v2, v3, v4, idx_hbm, w_hbm, out_hbm,
             idx_v, w_v, rows_v, out_v, semA, semB, semOA, semOB):
        c = lax.axis_index("c")
        s = lax.axis_index("s")
        wid = s * NC + c
        vols = (v1, v2, v3, v4)
        sems = (semA, semB)
        osems = (semOA, semOB)

        for l in range(4):
            pltpu.sync_copy(idx_hbm.at[wid, l], idx_v)
            pltpu.sync_copy(w_hbm.at[wid, l], w_v)

            def fire(ch, b, l=l):
                for k in range(8):
                    pltpu.async_copy(
                        vols[l].at[idx_v.at[pl.ds((ch * 8 + k) * PCHUNK,
                                                  PCHUNK)]],
                        rows_v.at[b, pl.ds(k * PCHUNK, PCHUNK)], sems[b])

            def drain(ch, b, l=l):
                for k in range(8):
                    pltpu.make_async_copy(
                        vols[l].at[idx_v.at[pl.ds((ch * 8 + k) * PCHUNK,
                                                  PCHUNK)]],
                        rows_v.at[b, pl.ds(k * PCHUNK, PCHUNK)],
                        sems[b]).wait()

            def compute(ch, b, l=l, first=False):
                def pair(pr, _):
                    rbase = pr * 16
                    wv = w_v[ch * 64 + pr, pl.ds(0, 16)]
                    for pt in range(2):
                        for half in range(2):
                            acc = wv[pt * 8] * rows_v[
                                b, rbase + pt * 8, pl.ds(half * 16, 16)]
                            for t in range(1, 8):
                                acc = acc + wv[pt * 8 + t] * rows_v[
                                    b, rbase + pt * 8 + t, pl.ds(half * 16, 16)]
                            out_v[b, pr * 2 + pt, pl.ds(half * 16, 16)] = acc
                    return 0
                base = (wid * NCHUNKS + ch) * PCHUNK
                dst = out_hbm.at[pl.ds(base, PCHUNK), pl.ds(l * 32, 32)]
                if not first:
                    pltpu.make_async_copy(out_v.at[b], dst, osems[b]).wait()
                lax.fori_loop(0, PCHUNK // 2, pair, 0)
                pltpu.async_copy(out_v.at[b], dst, osems[b])

            fire(0, 0)
            fire(1, 1)
            drain(0, 0)
            compute(0, 0, first=True)
            fire(2, 0)
            drain(1, 1)
            compute(1, 1, first=True)

            def pair(ii, _, l=l):
                ch0 = 2 * ii
                fire(ch0 + 1, 1)
                drain(ch0, 0)
                compute(ch0, 0)
                fire(ch0 + 2, 0)
                drain(ch0 + 1, 1)
                compute(ch0 + 1, 1)
                return 0

            lax.fori_loop(1, (NCHUNKS - 1) // 2, pair, 0)
            drain(NCHUNKS - 1, 0)
            compute(NCHUNKS - 1, 0)
            for b in range(2):
                dst = out_hbm.at[pl.ds(wid * NCHUNKS * PCHUNK, PCHUNK),
                                 pl.ds(l * 32, 32)]
                pltpu.make_async_copy(out_v.at[b], dst, osems[b]).wait()

    return body(l1f, l2f, l3f, l4f, idx, w8)


def _sample3d(vol, grid, dims):
    C = vol.shape[-1]
    D, H, W = dims[0], dims[1], dims[2]
    x = (grid[:, 0] + 1.0) * W * 0.5 - 0.5
    y = (grid[:, 1] + 1.0) * H * 0.5 - 0.5
    z = (grid[:, 2] + 1.0) * D * 0.5 - 0.5
    x0 = jnp.floor(x); y0 = jnp.floor(y); z0 = jnp.floor(z)
    wx = x - x0; wy = y - y0; wz = z - z0
    out = jnp.zeros((grid.shape[0], C), vol.dtype)
    for dz in (0, 1):
        for dy in (0, 1):
            for dx in (0, 1):
                zi = jnp.clip(z0 + dz, 0, D - 1).astype(jnp.int32)
                yi = jnp.clip(y0 + dy, 0, H - 1).astype(jnp.int32)
                xi = jnp.clip(x0 + dx, 0, W - 1).astype(jnp.int32)
                w = ((wz if dz else 1 - wz) * (wy if dy else 1 - wy) * (wx if dx else 1 - wx))[:, None]
                out = out + w * vol[zi, yi, xi]
    return out


def kernel(features, cnl_verts, canonical_pts, W1, b1, W2, b2, W3, b3, K1, K2, K3, K4):
    min_xyz = jnp.min(cnl_verts, axis=1) - 0.05
    max_xyz = jnp.max(cnl_verts, axis=1) + 0.05
    min_dhw = min_xyz[:, jnp.array([2, 1, 0])]
    max_dhw = max_xyz[:, jnp.array([2, 1, 0])]
    dhw = cnl_verts[..., jnp.array([2, 1, 0])]
    coord = jnp.round((dhw - min_dhw[:, None]) / VOXEL).astype(jnp.int32)
    out_sh = jnp.ceil((max_dhw - min_dhw) / VOXEL).astype(jnp.int32)
    out_sh = (out_sh | 31) + 1
    out_sh = jnp.max(out_sh, axis=0)

    coordT = jnp.pad(coord.reshape(NV, 3), ((0, VP - NV), (0, 0))).T
    s_rows, didx, sidx = _prep(features, coordT, W1, b1, W2, b2, W3, b3, K1)

    zeros_hbm = jnp.zeros((CHUNK, 32), jnp.float32)
    l1_rows = _scatter_sc(s_rows, didx, sidx, zeros_hbm)
    l1 = l1_rows.reshape(L1DEP, L1HW, L1HW, 32)

    l2 = _conv_level(l1, K2, 52, 24, 26, 26)
    l3 = _conv_level(l2, K3, 26, 12, 14, 14)
    l4 = _conv_level(l3, K4, 14, 6, 6, 6)

    ptsT = jnp.pad(canonical_pts.reshape(NP, 3), ((0, NPP - NP), (0, 0))).T
    osh_f = jnp.asarray(out_sh, jnp.float32)
    scal = jnp.concatenate([min_xyz[0], osh_f[jnp.array([2, 1, 0])],
                            jnp.zeros((2,), jnp.float32)]).reshape(1, 8)
    idx, w8 = _ptprep(ptsT, scal)
    idx = jnp.transpose(idx, (0, 1, 3, 2)).reshape(NTILES, 4,
                                                   NCHUNKS * PCHUNK * 8)
    w8 = jnp.transpose(w8, (0, 1, 3, 2)).reshape(NTILES, 4,
                                                 NCHUNKS * PCHUNK // 2, 16)
    out = _sample_sc(l1_rows, l2.reshape(-1, 32), l3.reshape(-1, 32),
                     l4.reshape(-1, 32), idx, w8)
    return out[:NP][None]
